# baseline (speedup 1.0000x reference)
import numpy as np
import jax
import jax.numpy as jnp
from jax.experimental import pallas as pl
from jax.experimental.pallas import tpu as pltpu

_B, _T, _NB = 64, 50, 360
_HID = 64
_LH = 192
_G = 32
_N = _G * _NB


def _gcn_kernel(scan_ref, coef_ref, w0t_ref, b0_ref, w1t_ref, b1_ref,
                w2t_ref, b2_ref, fwt_ref, fb_ref, pool_ref, out_ref):
    a = coef_ref[0:1, :]
    cl = coef_ref[1:2, :]
    cr = coef_ref[2:3, :]
    sa = coef_ref[3:4, :]

    def stencil(y):
        return (cl * pltpu.roll(y, 1, axis=1)
                + cr * pltpu.roll(y, _N - 1, axis=1) + a * y)

    scan = scan_ref[0]
    ss = stencil(scan)
    x = jnp.maximum(
        w0t_ref[:, 0:1] * ss + w0t_ref[:, 1:2] * sa + b0_ref[...], 0.0)
    x = jnp.maximum(
        stencil(jnp.dot(w1t_ref[...], x, preferred_element_type=jnp.float32))
        + b1_ref[...], 0.0)
    x = jnp.maximum(
        stencil(jnp.dot(w2t_ref[...], x, preferred_element_type=jnp.float32))
        + b2_ref[...], 0.0)
    g = jnp.dot(x, pool_ref[...], preferred_element_type=jnp.float32)
    out_ref[0] = jnp.maximum(
        jnp.dot(fwt_ref[...], g, preferred_element_type=jnp.float32)
        + fb_ref[...], 0.0)


def _lstm_kernel(xs_ref, wi0_ref, wh0_ref, b0_ref, wi1_ref, wh1_ref, b1_ref,
                 wi2_ref, wh2_ref, b2_ref, fw_ref, fb_ref, out_ref):
    def cell(x, h, c, wi_ref, wh_ref, b_ref):
        gates = (jnp.dot(x, wi_ref[...], preferred_element_type=jnp.float32)
                 + jnp.dot(h, wh_ref[...], preferred_element_type=jnp.float32)
                 + b_ref[...])
        i = jax.nn.sigmoid(gates[:, 0 * _LH:1 * _LH])
        f = jax.nn.sigmoid(gates[:, 1 * _LH:2 * _LH])
        g = jnp.tanh(gates[:, 2 * _LH:3 * _LH])
        o = jax.nn.sigmoid(gates[:, 3 * _LH:4 * _LH])
        c2 = f * c + i * g
        return o * jnp.tanh(c2), c2

    def step(t, carry):
        h0, c0, h1, c1, h2, c2 = carry
        x = xs_ref[t]
        h0, c0 = cell(x, h0, c0, wi0_ref, wh0_ref, b0_ref)
        h1, c1 = cell(h0, h1, c1, wi1_ref, wh1_ref, b1_ref)
        h2, c2 = cell(h1, h2, c2, wi2_ref, wh2_ref, b2_ref)
        return (h0, c0, h1, c1, h2, c2)

    z = jnp.zeros((_B, _LH), jnp.float32)
    carry = jax.lax.fori_loop(0, _T, step, (z, z, z, z, z, z))
    out_ref[...] = (
        jnp.dot(carry[4], fw_ref[...], preferred_element_type=jnp.float32)
        + fb_ref[...])


_POOL = np.kron(np.eye(_G, dtype=np.float32),
                np.full((_NB, 1), 1.0 / _NB, dtype=np.float32))


def _make_coef():
    deg = np.full((_NB,), 3.0, np.float32)
    deg[0] = deg[-1] = 2.0
    d = deg.astype(np.float32) ** -0.5
    a = d * d
    cl = np.zeros((_NB,), np.float32)
    cl[1:] = d[:-1] * d[1:]
    cr = np.zeros((_NB,), np.float32)
    cr[:-1] = d[:-1] * d[1:]
    ang = np.linspace(-np.pi, np.pi, _NB, dtype=np.float32)
    sa = np.zeros((_NB,), np.float32)
    sa[1:] += cl[1:] * ang[:-1]
    sa[:-1] += cr[:-1] * ang[1:]
    sa += a * ang
    rows = np.stack([a, cl, cr, sa]).astype(np.float32)
    return np.tile(rows, (1, _G))


_COEF = _make_coef()


def kernel(data, gcn_w0, gcn_b0, gcn_w1, gcn_b1, gcn_w2, gcn_b2, fcg_w, fcg_b,
           w_ih0, w_hh0, b_ih0, b_hh0, w_ih1, w_hh1, b_ih1, b_hh1,
           w_ih2, w_hh2, b_ih2, b_hh2, fco_w, fco_b):
    n_graphs = _B * _T
    n_tiles = n_graphs // _G
    scan_rows = data.reshape(n_tiles, 1, _N)
    wspec = lambda shp: pl.BlockSpec(shp, lambda i: (0, 0))
    embt = pl.pallas_call(
        _gcn_kernel,
        grid=(n_tiles,),
        in_specs=[
            pl.BlockSpec((1, 1, _N), lambda i: (i, 0, 0)),
            wspec((4, _N)),
            wspec((_HID, 2)), wspec((_HID, 1)),
            wspec((_HID, _HID)), wspec((_HID, 1)),
            wspec((_HID, _HID)), wspec((_HID, 1)),
            wspec((_HID, _HID)), wspec((_HID, 1)),
            wspec((_N, _G)),
        ],
        out_specs=pl.BlockSpec((1, _HID, _G), lambda i: (i, 0, 0)),
        out_shape=jax.ShapeDtypeStruct((n_tiles, _HID, _G), jnp.float32),
    )(scan_rows, jnp.asarray(_COEF), gcn_w0.T, gcn_b0.reshape(-1, 1),
      gcn_w1.T, gcn_b1.reshape(-1, 1), gcn_w2.T, gcn_b2.reshape(-1, 1),
      fcg_w.T, fcg_b.reshape(-1, 1), jnp.asarray(_POOL))

    emb = embt.transpose(0, 2, 1).reshape(_B, _T, _HID)
    xs = emb.transpose(1, 0, 2)
    out = pl.pallas_call(
        _lstm_kernel,
        out_shape=jax.ShapeDtypeStruct((_B, 3), jnp.float32),
    )(xs,
      w_ih0.T, w_hh0.T, (b_ih0 + b_hh0).reshape(1, -1),
      w_ih1.T, w_hh1.T, (b_ih1 + b_hh1).reshape(1, -1),
      w_ih2.T, w_hh2.T, (b_ih2 + b_hh2).reshape(1, -1),
      fco_w, fco_b.reshape(1, -1))
    return out


# speedup vs baseline: 1.0750x; 1.0750x over previous
import numpy as np
import jax
import jax.numpy as jnp
from jax.experimental import pallas as pl
from jax.experimental.pallas import tpu as pltpu

_B, _T, _NB = 64, 50, 360
_HID = 64
_LH = 192
_G = 64
_N = _G * _NB


def _gcn_kernel(scan_ref, coef_ref, w0t_ref, b0_ref, w1t_ref, b1_ref,
                w2t_ref, b2_ref, fwt_ref, fb_ref, pool_ref, out_ref):
    a = coef_ref[0:1, :]
    cl = coef_ref[1:2, :]
    cr = coef_ref[2:3, :]
    sa = coef_ref[3:4, :]

    def stencil(y):
        return (cl * pltpu.roll(y, 1, axis=1)
                + cr * pltpu.roll(y, _N - 1, axis=1) + a * y)

    scan = scan_ref[0]
    ss = stencil(scan)
    x = jnp.maximum(
        w0t_ref[:, 0:1] * ss + w0t_ref[:, 1:2] * sa + b0_ref[...], 0.0)
    x = jnp.maximum(
        stencil(jnp.dot(w1t_ref[...], x, preferred_element_type=jnp.float32))
        + b1_ref[...], 0.0)
    x = jnp.maximum(
        stencil(jnp.dot(w2t_ref[...], x, preferred_element_type=jnp.float32))
        + b2_ref[...], 0.0)
    g = jnp.dot(x, pool_ref[...], preferred_element_type=jnp.float32)
    out_ref[0] = jnp.maximum(
        jnp.dot(fwt_ref[...], g, preferred_element_type=jnp.float32)
        + fb_ref[...], 0.0)


def _lstm_kernel(xs_ref, wi0_ref, wh0_ref, b0_ref, wi1_ref, wh1_ref, b1_ref,
                 wi2_ref, wh2_ref, b2_ref, fw_ref, fb_ref, out_ref):
    def cell(x, h, c, wi_ref, wh_ref, b_ref):
        gates = (jnp.dot(x, wi_ref[...], preferred_element_type=jnp.float32)
                 + jnp.dot(h, wh_ref[...], preferred_element_type=jnp.float32)
                 + b_ref[...])
        i = jax.nn.sigmoid(gates[:, 0 * _LH:1 * _LH])
        f = jax.nn.sigmoid(gates[:, 1 * _LH:2 * _LH])
        g = jnp.tanh(gates[:, 2 * _LH:3 * _LH])
        o = jax.nn.sigmoid(gates[:, 3 * _LH:4 * _LH])
        c2 = f * c + i * g
        return o * jnp.tanh(c2), c2

    def step(t, carry):
        h0, c0, h1, c1, h2, c2 = carry
        x = xs_ref[t]
        h0, c0 = cell(x, h0, c0, wi0_ref, wh0_ref, b0_ref)
        h1, c1 = cell(h0, h1, c1, wi1_ref, wh1_ref, b1_ref)
        h2, c2 = cell(h1, h2, c2, wi2_ref, wh2_ref, b2_ref)
        return (h0, c0, h1, c1, h2, c2)

    z = jnp.zeros((_B, _LH), jnp.float32)
    carry = jax.lax.fori_loop(0, _T, step, (z, z, z, z, z, z))
    out_ref[...] = (
        jnp.dot(carry[4], fw_ref[...], preferred_element_type=jnp.float32)
        + fb_ref[...])


_POOL = np.kron(np.eye(_G, dtype=np.float32),
                np.full((_NB, 1), 1.0 / _NB, dtype=np.float32))


def _make_coef():
    deg = np.full((_NB,), 3.0, np.float32)
    deg[0] = deg[-1] = 2.0
    d = deg.astype(np.float32) ** -0.5
    a = d * d
    cl = np.zeros((_NB,), np.float32)
    cl[1:] = d[:-1] * d[1:]
    cr = np.zeros((_NB,), np.float32)
    cr[:-1] = d[:-1] * d[1:]
    ang = np.linspace(-np.pi, np.pi, _NB, dtype=np.float32)
    sa = np.zeros((_NB,), np.float32)
    sa[1:] += cl[1:] * ang[:-1]
    sa[:-1] += cr[:-1] * ang[1:]
    sa += a * ang
    rows = np.stack([a, cl, cr, sa]).astype(np.float32)
    return np.tile(rows, (1, _G))


_COEF = _make_coef()


def kernel(data, gcn_w0, gcn_b0, gcn_w1, gcn_b1, gcn_w2, gcn_b2, fcg_w, fcg_b,
           w_ih0, w_hh0, b_ih0, b_hh0, w_ih1, w_hh1, b_ih1, b_hh1,
           w_ih2, w_hh2, b_ih2, b_hh2, fco_w, fco_b):
    n_graphs = _B * _T
    n_tiles = n_graphs // _G
    scan_rows = data.reshape(n_tiles, 1, _N)
    wspec = lambda shp: pl.BlockSpec(shp, lambda i: (0, 0))
    embt = pl.pallas_call(
        _gcn_kernel,
        grid=(n_tiles,),
        in_specs=[
            pl.BlockSpec((1, 1, _N), lambda i: (i, 0, 0)),
            wspec((4, _N)),
            wspec((_HID, 2)), wspec((_HID, 1)),
            wspec((_HID, _HID)), wspec((_HID, 1)),
            wspec((_HID, _HID)), wspec((_HID, 1)),
            wspec((_HID, _HID)), wspec((_HID, 1)),
            wspec((_N, _G)),
        ],
        out_specs=pl.BlockSpec((1, _HID, _G), lambda i: (i, 0, 0)),
        out_shape=jax.ShapeDtypeStruct((n_tiles, _HID, _G), jnp.float32),
    )(scan_rows, jnp.asarray(_COEF), gcn_w0.T, gcn_b0.reshape(-1, 1),
      gcn_w1.T, gcn_b1.reshape(-1, 1), gcn_w2.T, gcn_b2.reshape(-1, 1),
      fcg_w.T, fcg_b.reshape(-1, 1), jnp.asarray(_POOL))

    emb = embt.transpose(0, 2, 1).reshape(_B, _T, _HID)
    xs = emb.transpose(1, 0, 2)
    out = pl.pallas_call(
        _lstm_kernel,
        out_shape=jax.ShapeDtypeStruct((_B, 3), jnp.float32),
    )(xs,
      w_ih0.T, w_hh0.T, (b_ih0 + b_hh0).reshape(1, -1),
      w_ih1.T, w_hh1.T, (b_ih1 + b_hh1).reshape(1, -1),
      w_ih2.T, w_hh2.T, (b_ih2 + b_hh2).reshape(1, -1),
      fco_w, fco_b.reshape(1, -1))
    return out
